# baseline (speedup 1.0000x reference)
import functools

import jax
import jax.numpy as jnp
from jax import lax
from jax.experimental import pallas as pl
from jax.experimental.pallas import tpu as pltpu
from jax.experimental.pallas import tpu_sc as plsc

_NW = 32


def _sc_gather(xyzTf, idx2, dist2):
    B, N3 = xyzTf.shape
    N = N3 // 3
    NK = idx2.shape[1]
    CH = NK // _NW
    NV = CH // 16
    mesh = plsc.VectorSubcoreMesh(core_axis_name="c", subcore_axis_name="s")

    @functools.partial(
        pl.kernel,
        mesh=mesh,
        compiler_params=pltpu.CompilerParams(needs_layout_passes=False),
        out_type=jax.ShapeDtypeStruct((B, 8, NK), jnp.float32),
        scratch_types=[
            pltpu.VMEM((3 * N,), jnp.float32),
            pltpu.VMEM((CH,), jnp.int32),
            pltpu.VMEM((8 * CH,), jnp.float32),
        ],
    )
    def k(xyzT_hbm, idx_hbm, dist_hbm, out_hbm, xyz_v, idx_v, feat_v):
        wid = lax.axis_index("s") * 2 + lax.axis_index("c")
        base = wid * CH
        ones = jnp.ones((16,), jnp.float32)

        def init_ones(i, carry):
            feat_v[pl.ds(7 * CH + i * 16, 16)] = ones
            return carry

        lax.fori_loop(0, NV, init_ones, 0)

        for b in range(B):
            pltpu.sync_copy(xyzT_hbm.at[b], xyz_v)
            pltpu.sync_copy(idx_hbm.at[b, pl.ds(base, CH)], idx_v)
            pltpu.sync_copy(dist_hbm.at[b, pl.ds(base, CH)],
                            feat_v.at[pl.ds(3 * CH, CH)])

            def body(i, carry):
                iv = idx_v[pl.ds(i * 16, 16)]
                nvec = jnp.broadcast_to(wid * NV + i, (16,)).astype(jnp.int32)
                for c in range(3):
                    g = plsc.load_gather(xyz_v, [iv + (c * N)])
                    feat_v[pl.ds(c * CH + i * 16, 16)] = g
                    gc = plsc.load_gather(xyz_v, [nvec + (c * N)])
                    feat_v[pl.ds((4 + c) * CH + i * 16, 16)] = gc
                return carry

            lax.fori_loop(0, NV, body, 0)
            for c in range(8):
                pltpu.sync_copy(feat_v.at[pl.ds(c * CH, CH)],
                                out_hbm.at[b, c, pl.ds(base, CH)])

    return k(xyzTf, idx2, dist2)


def _tc_body(feat_ref, w1_ref, w2_ref, b2_ref, out_ref):
    x = feat_ref[0]
    h = lax.dot_general(x, w1_ref[...], (((0,), (0,)), ((), ())),
                        preferred_element_type=jnp.float32)
    h = jnp.maximum(h, 0.0)
    out_ref[0] = jnp.dot(h, w2_ref[...],
                         preferred_element_type=jnp.float32) + b2_ref[...]


def _tc_mlp(feat, W1e, W2, b2row):
    B, _, NK = feat.shape
    dim = W2.shape[1]
    MB = 8192
    grid = (B, NK // MB)
    return pl.pallas_call(
        _tc_body,
        grid=grid,
        in_specs=[
            pl.BlockSpec((1, 8, MB), lambda b, j: (b, 0, j)),
            pl.BlockSpec((8, dim), lambda b, j: (0, 0)),
            pl.BlockSpec((dim, dim), lambda b, j: (0, 0)),
            pl.BlockSpec((1, dim), lambda b, j: (0, 0)),
        ],
        out_specs=pl.BlockSpec((1, MB, dim), lambda b, j: (b, j, 0)),
        out_shape=jax.ShapeDtypeStruct((B, NK, dim), jnp.float32),
    )(feat, W1e, W2, b2row)


def kernel(xyz, idx, dist, W1, b1, W2, b2, num_neighbors=16):
    B, N, K = idx.shape
    NK = N * K
    xyzTf = jnp.transpose(xyz, (0, 2, 1)).reshape(B, 3 * N)
    idx2 = idx.reshape(B, NK)
    dist2 = dist.reshape(B, NK)
    A = W1[0:3] + W1[6:9]
    Bm = W1[3:6] - W1[6:9]
    W1e = jnp.concatenate([Bm, W1[9:10], A, b1[None, :]], axis=0)
    feat = _sc_gather(xyzTf, idx2, dist2)
    out = _tc_mlp(feat, W1e, W2, b2[None, :])
    return out.reshape(B, N, K, -1)


# speedup vs baseline: 2.3055x; 2.3055x over previous
import functools

import jax
import jax.numpy as jnp
from jax import lax
from jax.experimental import pallas as pl
from jax.experimental.pallas import tpu as pltpu
from jax.experimental.pallas import tpu_sc as plsc

_NW = 32


def _sc_gather(xyzTf, idxP, distP):
    B, N3 = xyzTf.shape
    N = N3 // 3
    K = idxP.shape[1]
    KPW = (B * K) // _NW
    NV = N // 16
    mesh = plsc.VectorSubcoreMesh(core_axis_name="c", subcore_axis_name="s")

    @functools.partial(
        pl.kernel,
        mesh=mesh,
        compiler_params=pltpu.CompilerParams(needs_layout_passes=False),
        out_type=jax.ShapeDtypeStruct((B, K, 4 * N), jnp.float32),
        scratch_types=[
            pltpu.VMEM((3 * N,), jnp.float32),
            pltpu.VMEM((N,), jnp.int32),
            pltpu.VMEM((4 * N,), jnp.float32),
        ],
    )
    def k(xyzT_hbm, idx_hbm, dist_hbm, out_hbm, xyz_v, idx_v, f4_v):
        wid = lax.axis_index("s") * 2 + lax.axis_index("c")
        b = wid // (_NW // B)
        k0 = (wid % (_NW // B)) * KPW
        pltpu.sync_copy(xyzT_hbm.at[b], xyz_v)
        for dk in range(KPW):
            kk = k0 + dk
            pltpu.sync_copy(idx_hbm.at[b, kk], idx_v)
            pltpu.sync_copy(dist_hbm.at[b, kk], f4_v.at[pl.ds(3 * N, N)])

            def body(i, carry):
                iv = idx_v[pl.ds(i * 16, 16)]
                for c in range(3):
                    g = plsc.load_gather(xyz_v, [iv + (c * N)])
                    f4_v[pl.ds(c * N + i * 16, 16)] = g
                return carry

            lax.fori_loop(0, NV, body, 0)
            pltpu.sync_copy(f4_v, out_hbm.at[b, kk])

    return k(xyzTf, idxP, distP)


def _cct_body(xyz_ref, at_ref, b1_ref, out_ref):
    out_ref[0] = jnp.dot(at_ref[...], xyz_ref[0],
                         preferred_element_type=jnp.float32) + b1_ref[...]


def _cct(xyzT, AT, b1col):
    B, _, N = xyzT.shape
    dim = AT.shape[0]
    return pl.pallas_call(
        _cct_body,
        grid=(B,),
        in_specs=[
            pl.BlockSpec((1, 3, N), lambda b: (b, 0, 0)),
            pl.BlockSpec((dim, 3), lambda b: (0, 0)),
            pl.BlockSpec((dim, 1), lambda b: (0, 0)),
        ],
        out_specs=pl.BlockSpec((1, dim, N), lambda b: (b, 0, 0)),
        out_shape=jax.ShapeDtypeStruct((B, dim, N), jnp.float32),
    )(xyzT, AT, b1col)


def _mlp_body(f4_ref, cct_ref, wn_ref, w2_ref, b2_ref, out_ref):
    ht = jnp.dot(wn_ref[...], f4_ref[0, 0],
                 preferred_element_type=jnp.float32) + cct_ref[0]
    ht = jnp.maximum(ht, 0.0)
    out_ref[0, 0] = jnp.dot(w2_ref[...], ht,
                            preferred_element_type=jnp.float32) + b2_ref[...]


def _mlp(F4, CcT, Wxn4T, W2T, b2col):
    B, K, N4 = F4.shape
    N = N4 // 4
    dim = W2T.shape[0]
    F4 = F4.reshape(B, K, 4, N)
    return pl.pallas_call(
        _mlp_body,
        grid=(B, K),
        in_specs=[
            pl.BlockSpec((1, 1, 4, N), lambda b, k: (b, k, 0, 0)),
            pl.BlockSpec((1, dim, N), lambda b, k: (b, 0, 0)),
            pl.BlockSpec((dim, 4), lambda b, k: (0, 0)),
            pl.BlockSpec((dim, dim), lambda b, k: (0, 0)),
            pl.BlockSpec((dim, 1), lambda b, k: (0, 0)),
        ],
        out_specs=pl.BlockSpec((1, 1, dim, N), lambda b, k: (b, k, 0, 0)),
        out_shape=jax.ShapeDtypeStruct((B, K, dim, N), jnp.float32),
    )(F4, CcT, Wxn4T, W2T, b2col)


def kernel(xyz, idx, dist, W1, b1, W2, b2, num_neighbors=16):
    B, N, K = idx.shape
    xyzT = jnp.transpose(xyz, (0, 2, 1))
    idxP = jnp.transpose(idx, (0, 2, 1))
    distP = jnp.transpose(dist, (0, 2, 1))
    A = W1[0:3] + W1[6:9]
    Bm = W1[3:6] - W1[6:9]
    AT = A.T
    Wxn4T = jnp.concatenate([Bm.T, W1[9:10].T], axis=1)
    F4 = _sc_gather(xyzT.reshape(B, 3 * N), idxP, distP)
    CcT = _cct(xyzT, AT, b1[:, None])
    outP = _mlp(F4, CcT, Wxn4T, W2.T, b2[:, None])
    return jnp.transpose(outP, (0, 3, 1, 2))


# speedup vs baseline: 2.4711x; 1.0718x over previous
import functools

import jax
import jax.numpy as jnp
from jax import lax
from jax.experimental import pallas as pl
from jax.experimental.pallas import tpu as pltpu
from jax.experimental.pallas import tpu_sc as plsc

_NW = 32


def _sc_gather(xyzTf, idxP, distP, b0, bh):
    B, N3 = xyzTf.shape
    N = N3 // 3
    K = idxP.shape[1]
    KPW = (bh * K) // _NW
    WPB = _NW // bh
    NV = N // 16
    mesh = plsc.VectorSubcoreMesh(core_axis_name="c", subcore_axis_name="s")

    @functools.partial(
        pl.kernel,
        mesh=mesh,
        compiler_params=pltpu.CompilerParams(needs_layout_passes=False),
        out_type=jax.ShapeDtypeStruct((bh, K, 4 * N), jnp.float32),
        scratch_types=[
            pltpu.VMEM((3 * N,), jnp.float32),
            pltpu.VMEM((N,), jnp.int32),
            pltpu.VMEM((4 * N,), jnp.float32),
        ],
    )
    def k(xyzT_hbm, idx_hbm, dist_hbm, out_hbm, xyz_v, idx_v, f4_v):
        wid = lax.axis_index("s") * 2 + lax.axis_index("c")
        bl = wid // WPB
        k0 = (wid % WPB) * KPW
        pltpu.sync_copy(xyzT_hbm.at[b0 + bl], xyz_v)
        for dk in range(KPW):
            kk = k0 + dk
            pltpu.sync_copy(idx_hbm.at[b0 + bl, kk], idx_v)
            pltpu.sync_copy(dist_hbm.at[b0 + bl, kk], f4_v.at[pl.ds(3 * N, N)])

            def body(i, carry):
                iv = idx_v[pl.ds(i * 16, 16)]
                for c in range(3):
                    g = plsc.load_gather(xyz_v, [iv + (c * N)])
                    f4_v[pl.ds(c * N + i * 16, 16)] = g
                return carry

            lax.fori_loop(0, NV, body, 0)
            pltpu.sync_copy(f4_v, out_hbm.at[bl, kk])

    return k(xyzTf, idxP, distP)


def _cct_body(xyz_ref, at_ref, b1_ref, out_ref):
    out_ref[0] = jnp.dot(at_ref[...], xyz_ref[0],
                         preferred_element_type=jnp.float32) + b1_ref[...]


def _cct(xyzT, AT, b1col):
    B, _, N = xyzT.shape
    dim = AT.shape[0]
    return pl.pallas_call(
        _cct_body,
        grid=(B,),
        in_specs=[
            pl.BlockSpec((1, 3, N), lambda b: (b, 0, 0)),
            pl.BlockSpec((dim, 3), lambda b: (0, 0)),
            pl.BlockSpec((dim, 1), lambda b: (0, 0)),
        ],
        out_specs=pl.BlockSpec((1, dim, N), lambda b: (b, 0, 0)),
        out_shape=jax.ShapeDtypeStruct((B, dim, N), jnp.float32),
    )(xyzT, AT, b1col)


def _mlp_body(f4_ref, cct_ref, wn_ref, w2_ref, b2_ref, out_ref):
    ht = jnp.dot(wn_ref[...], f4_ref[0, 0],
                 preferred_element_type=jnp.float32) + cct_ref[0]
    ht = jnp.maximum(ht, 0.0)
    out_ref[0, 0] = jnp.dot(w2_ref[...], ht,
                            preferred_element_type=jnp.float32) + b2_ref[...]


def _mlp_alias_body(f4_ref, cct_ref, wn_ref, w2_ref, b2_ref, buf_ref, out_ref):
    _mlp_body(f4_ref, cct_ref, wn_ref, w2_ref, b2_ref, out_ref)


def _mlp(F4h, CcT, Wxn4T, W2T, b2col, b0, buf=None):
    bh, K, N4 = F4h.shape
    N = N4 // 4
    B = CcT.shape[0]
    dim = W2T.shape[0]
    F4h = F4h.reshape(bh, K, 4, N)
    in_specs = [
        pl.BlockSpec((1, 1, 4, N), lambda b, k: (b, k, 0, 0)),
        pl.BlockSpec((1, dim, N), lambda b, k: (b0 + b, 0, 0)),
        pl.BlockSpec((dim, 4), lambda b, k: (0, 0)),
        pl.BlockSpec((dim, dim), lambda b, k: (0, 0)),
        pl.BlockSpec((dim, 1), lambda b, k: (0, 0)),
    ]
    args = [F4h, CcT, Wxn4T, W2T, b2col]
    kwargs = {}
    body = _mlp_body
    if buf is not None:
        in_specs.append(pl.BlockSpec(memory_space=pl.MemorySpace.ANY))
        args.append(buf)
        kwargs["input_output_aliases"] = {5: 0}
        body = _mlp_alias_body
    return pl.pallas_call(
        body,
        grid=(bh, K),
        in_specs=in_specs,
        out_specs=pl.BlockSpec((1, 1, dim, N), lambda b, k: (b0 + b, k, 0, 0)),
        out_shape=jax.ShapeDtypeStruct((B, K, dim, N), jnp.float32),
        **kwargs,
    )(*args)


def kernel(xyz, idx, dist, W1, b1, W2, b2, num_neighbors=16):
    B, N, K = idx.shape
    xyzT = jnp.transpose(xyz, (0, 2, 1))
    xyzTf = xyzT.reshape(B, 3 * N)
    idxP = jnp.transpose(idx, (0, 2, 1))
    distP = jnp.transpose(dist, (0, 2, 1))
    A = W1[0:3] + W1[6:9]
    Bm = W1[3:6] - W1[6:9]
    AT = A.T
    Wxn4T = jnp.concatenate([Bm.T, W1[9:10].T], axis=1)
    b1col = b1[:, None]
    b2col = b2[:, None]
    W2T = W2.T
    bh = B // 2
    F4a = _sc_gather(xyzTf, idxP, distP, 0, bh)
    F4b = _sc_gather(xyzTf, idxP, distP, bh, bh)
    CcT = _cct(xyzT, AT, b1col)
    buf = _mlp(F4a, CcT, Wxn4T, W2T, b2col, 0)
    outP = _mlp(F4b, CcT, Wxn4T, W2T, b2col, bh, buf)
    return jnp.transpose(outP, (0, 3, 1, 2))


# speedup vs baseline: 2.4769x; 1.0023x over previous
import functools

import jax
import jax.numpy as jnp
from jax import lax
from jax.experimental import pallas as pl
from jax.experimental.pallas import tpu as pltpu
from jax.experimental.pallas import tpu_sc as plsc

_NW = 32


def _sc_gather(xyzTf, idxP, distP, b0, bh):
    B, N3 = xyzTf.shape
    N = N3 // 3
    K = idxP.shape[1]
    KPW = (bh * K) // _NW
    WPB = _NW // bh
    NV = N // 16
    mesh = plsc.VectorSubcoreMesh(core_axis_name="c", subcore_axis_name="s")

    @functools.partial(
        pl.kernel,
        mesh=mesh,
        compiler_params=pltpu.CompilerParams(needs_layout_passes=False),
        out_type=jax.ShapeDtypeStruct((bh, K, 4 * N), jnp.float32),
        scratch_types=[
            pltpu.VMEM((3 * N,), jnp.float32),
            pltpu.VMEM((N,), jnp.int32),
            pltpu.VMEM((4 * N,), jnp.float32),
        ],
    )
    def k(xyzT_hbm, idx_hbm, dist_hbm, out_hbm, xyz_v, idx_v, f4_v):
        wid = lax.axis_index("s") * 2 + lax.axis_index("c")
        bl = wid // WPB
        k0 = (wid % WPB) * KPW
        pltpu.sync_copy(xyzT_hbm.at[b0 + bl], xyz_v)
        for dk in range(KPW):
            kk = k0 + dk
            pltpu.sync_copy(idx_hbm.at[b0 + bl, kk], idx_v)
            pltpu.sync_copy(dist_hbm.at[b0 + bl, kk], f4_v.at[pl.ds(3 * N, N)])

            def body(i, carry):
                iv = idx_v[pl.ds(i * 16, 16)]
                for c in range(3):
                    g = plsc.load_gather(xyz_v, [iv + (c * N)])
                    f4_v[pl.ds(c * N + i * 16, 16)] = g
                return carry

            lax.fori_loop(0, NV, body, 0)
            pltpu.sync_copy(f4_v, out_hbm.at[bl, kk])

    return k(xyzTf, idxP, distP)


def _cct_body(xyz_ref, at_ref, b1_ref, out_ref):
    out_ref[0] = jnp.dot(at_ref[...], xyz_ref[0],
                         preferred_element_type=jnp.float32) + b1_ref[...]


def _cct(xyzT, AT, b1col):
    B, _, N = xyzT.shape
    dim = AT.shape[0]
    return pl.pallas_call(
        _cct_body,
        grid=(B,),
        in_specs=[
            pl.BlockSpec((1, 3, N), lambda b: (b, 0, 0)),
            pl.BlockSpec((dim, 3), lambda b: (0, 0)),
            pl.BlockSpec((dim, 1), lambda b: (0, 0)),
        ],
        out_specs=pl.BlockSpec((1, dim, N), lambda b: (b, 0, 0)),
        out_shape=jax.ShapeDtypeStruct((B, dim, N), jnp.float32),
    )(xyzT, AT, b1col)


def _mlp_body(f4_ref, cct_ref, wn_ref, w2_ref, b2_ref, out_ref):
    ht = jnp.dot(wn_ref[...], f4_ref[0, 0],
                 preferred_element_type=jnp.float32) + cct_ref[0]
    ht = jnp.maximum(ht, 0.0).astype(jnp.bfloat16)
    out_ref[0, 0] = jnp.dot(w2_ref[...], ht,
                            preferred_element_type=jnp.float32) + b2_ref[...]


def _mlp_alias_body(f4_ref, cct_ref, wn_ref, w2_ref, b2_ref, buf_ref, out_ref):
    _mlp_body(f4_ref, cct_ref, wn_ref, w2_ref, b2_ref, out_ref)


def _mlp(F4h, CcT, Wxn4T, W2T, b2col, b0, buf=None):
    bh, K, N4 = F4h.shape
    N = N4 // 4
    B = CcT.shape[0]
    dim = W2T.shape[0]
    F4h = F4h.reshape(bh, K, 4, N)
    in_specs = [
        pl.BlockSpec((1, 1, 4, N), lambda b, k: (b, k, 0, 0)),
        pl.BlockSpec((1, dim, N), lambda b, k: (b0 + b, 0, 0)),
        pl.BlockSpec((dim, 4), lambda b, k: (0, 0)),
        pl.BlockSpec((dim, dim), lambda b, k: (0, 0)),
        pl.BlockSpec((dim, 1), lambda b, k: (0, 0)),
    ]
    args = [F4h, CcT, Wxn4T, W2T, b2col]
    kwargs = {}
    body = _mlp_body
    if buf is not None:
        in_specs.append(pl.BlockSpec(memory_space=pl.MemorySpace.ANY))
        args.append(buf)
        kwargs["input_output_aliases"] = {5: 0}
        body = _mlp_alias_body
    return pl.pallas_call(
        body,
        grid=(bh, K),
        in_specs=in_specs,
        out_specs=pl.BlockSpec((1, 1, dim, N), lambda b, k: (b0 + b, k, 0, 0)),
        out_shape=jax.ShapeDtypeStruct((B, K, dim, N), jnp.float32),
        **kwargs,
    )(*args)


def kernel(xyz, idx, dist, W1, b1, W2, b2, num_neighbors=16):
    B, N, K = idx.shape
    xyzT = jnp.transpose(xyz, (0, 2, 1))
    xyzTf = xyzT.reshape(B, 3 * N)
    idxP = jnp.transpose(idx, (0, 2, 1))
    distP = jnp.transpose(dist, (0, 2, 1))
    A = W1[0:3] + W1[6:9]
    Bm = W1[3:6] - W1[6:9]
    AT = A.T
    Wxn4T = jnp.concatenate([Bm.T, W1[9:10].T], axis=1)
    b1col = b1[:, None]
    b2col = b2[:, None]
    W2T = W2.T.astype(jnp.bfloat16)
    bh = B // 2
    F4a = _sc_gather(xyzTf, idxP, distP, 0, bh)
    F4b = _sc_gather(xyzTf, idxP, distP, bh, bh)
    CcT = _cct(xyzT, AT, b1col)
    buf = _mlp(F4a, CcT, Wxn4T, W2T, b2col, 0)
    outP = _mlp(F4b, CcT, Wxn4T, W2T, b2col, bh, buf)
    return jnp.transpose(outP, (0, 3, 1, 2))


# speedup vs baseline: 3.7185x; 1.5013x over previous
import functools

import jax
import jax.numpy as jnp
from jax import lax
from jax.experimental import pallas as pl
from jax.experimental.pallas import tpu as pltpu
from jax.experimental.pallas import tpu_sc as plsc

_NW = 32


def _sc_gather(xyzTf, idxP, distP, b0, bh):
    B, N3 = xyzTf.shape
    N = N3 // 3
    K = idxP.shape[1]
    KPW = (bh * K) // _NW
    WPB = _NW // bh
    NV = N // 16
    mesh = plsc.VectorSubcoreMesh(core_axis_name="c", subcore_axis_name="s")

    @functools.partial(
        pl.kernel,
        mesh=mesh,
        compiler_params=pltpu.CompilerParams(needs_layout_passes=False),
        out_type=jax.ShapeDtypeStruct((bh, K, 4 * N), jnp.float32),
        scratch_types=[
            pltpu.VMEM((3 * N,), jnp.float32),
            pltpu.VMEM((N,), jnp.int32),
            pltpu.VMEM((4 * N,), jnp.float32),
        ],
    )
    def k(xyzT_hbm, idx_hbm, dist_hbm, out_hbm, xyz_v, idx_v, f4_v):
        wid = lax.axis_index("s") * 2 + lax.axis_index("c")
        bl = wid // WPB
        k0 = (wid % WPB) * KPW
        pltpu.sync_copy(xyzT_hbm.at[b0 + bl], xyz_v)
        for dk in range(KPW):
            kk = k0 + dk
            pltpu.sync_copy(idx_hbm.at[b0 + bl, kk], idx_v)
            pltpu.sync_copy(dist_hbm.at[b0 + bl, kk], f4_v.at[pl.ds(3 * N, N)])

            def body(i, carry):
                iv = idx_v[pl.ds(i * 16, 16)]
                for c in range(3):
                    g = plsc.load_gather(xyz_v, [iv + (c * N)])
                    f4_v[pl.ds(c * N + i * 16, 16)] = g
                return carry

            lax.fori_loop(0, NV, body, 0)
            pltpu.sync_copy(f4_v, out_hbm.at[bl, kk])

    return k(xyzTf, idxP, distP)


def _cct_body(xyz_ref, at_ref, b1_ref, out_ref):
    out_ref[0] = jnp.dot(at_ref[...], xyz_ref[0],
                         preferred_element_type=jnp.float32) + b1_ref[...]


def _cct(xyzT, AT, b1col):
    B, _, N = xyzT.shape
    dim = AT.shape[0]
    return pl.pallas_call(
        _cct_body,
        grid=(B,),
        in_specs=[
            pl.BlockSpec((1, 3, N), lambda b: (b, 0, 0)),
            pl.BlockSpec((dim, 3), lambda b: (0, 0)),
            pl.BlockSpec((dim, 1), lambda b: (0, 0)),
        ],
        out_specs=pl.BlockSpec((1, dim, N), lambda b: (b, 0, 0)),
        out_shape=jax.ShapeDtypeStruct((B, dim, N), jnp.float32),
    )(xyzT, AT, b1col)


_KB = 4


def _mlp_body(f4_ref, cct_ref, wn_ref, w2_ref, b2_ref, out_ref):
    cct = cct_ref[0]
    for j in range(_KB):
        ht = jnp.dot(wn_ref[...], f4_ref[0, j],
                     preferred_element_type=jnp.float32) + cct
        ht = jnp.maximum(ht, 0.0).astype(jnp.bfloat16)
        out_ref[0, j] = jnp.dot(w2_ref[...], ht,
                                preferred_element_type=jnp.float32) + b2_ref[...]


def _mlp_alias_body(f4_ref, cct_ref, wn_ref, w2_ref, b2_ref, buf_ref, out_ref):
    _mlp_body(f4_ref, cct_ref, wn_ref, w2_ref, b2_ref, out_ref)


def _mlp(F4h, CcT, Wxn4T, W2T, b2col, b0, buf=None):
    bh, K, N4 = F4h.shape
    N = N4 // 4
    B = CcT.shape[0]
    dim = W2T.shape[0]
    F4h = F4h.reshape(bh, K, 4, N)
    in_specs = [
        pl.BlockSpec((1, _KB, 4, N), lambda b, k: (b, k, 0, 0)),
        pl.BlockSpec((1, dim, N), lambda b, k: (b0 + b, 0, 0)),
        pl.BlockSpec((dim, 4), lambda b, k: (0, 0)),
        pl.BlockSpec((dim, dim), lambda b, k: (0, 0)),
        pl.BlockSpec((dim, 1), lambda b, k: (0, 0)),
    ]
    args = [F4h, CcT, Wxn4T, W2T, b2col]
    kwargs = {}
    body = _mlp_body
    if buf is not None:
        in_specs.append(pl.BlockSpec(memory_space=pl.MemorySpace.ANY))
        args.append(buf)
        kwargs["input_output_aliases"] = {5: 0}
        body = _mlp_alias_body
    return pl.pallas_call(
        body,
        grid=(bh, K // _KB),
        in_specs=in_specs,
        out_specs=pl.BlockSpec((1, _KB, dim, N),
                               lambda b, k: (b0 + b, k, 0, 0)),
        out_shape=jax.ShapeDtypeStruct((B, K, dim, N), jnp.float32),
        **kwargs,
    )(*args)


def kernel(xyz, idx, dist, W1, b1, W2, b2, num_neighbors=16):
    B, N, K = idx.shape
    xyzT = jnp.transpose(xyz, (0, 2, 1))
    xyzTf = xyzT.reshape(B, 3 * N)
    idxP = jnp.transpose(idx, (0, 2, 1))
    distP = jnp.transpose(dist, (0, 2, 1))
    A = W1[0:3] + W1[6:9]
    Bm = W1[3:6] - W1[6:9]
    AT = A.T
    Wxn4T = jnp.concatenate([Bm.T, W1[9:10].T], axis=1)
    b1col = b1[:, None]
    b2col = b2[:, None]
    W2T = W2.T.astype(jnp.bfloat16)
    nsplit = 4
    bh = B // nsplit
    F4s = [_sc_gather(xyzTf, idxP, distP, s * bh, bh) for s in range(nsplit)]
    CcT = _cct(xyzT, AT, b1col)
    buf = _mlp(F4s[0], CcT, Wxn4T, W2T, b2col, 0)
    for s in range(1, nsplit):
        buf = _mlp(F4s[s], CcT, Wxn4T, W2T, b2col, s * bh, buf)
    return jnp.transpose(buf, (0, 3, 1, 2))


# speedup vs baseline: 3.9362x; 1.0585x over previous
import functools

import jax
import jax.numpy as jnp
from jax import lax
from jax.experimental import pallas as pl
from jax.experimental.pallas import tpu as pltpu
from jax.experimental.pallas import tpu_sc as plsc

_NW = 32


def _sc_gather(xyzTf, idxP, distP, b0, bh):
    B, N3 = xyzTf.shape
    N = N3 // 3
    K = idxP.shape[1]
    KPW = (bh * K) // _NW
    WPB = _NW // bh
    NV = N // 16
    mesh = plsc.VectorSubcoreMesh(core_axis_name="c", subcore_axis_name="s")

    @functools.partial(
        pl.kernel,
        mesh=mesh,
        compiler_params=pltpu.CompilerParams(needs_layout_passes=False),
        out_type=jax.ShapeDtypeStruct((bh, K, 4 * N), jnp.float32),
        scratch_types=[
            pltpu.VMEM((3 * N,), jnp.float32),
            pltpu.VMEM((N,), jnp.int32),
            pltpu.VMEM((4 * N,), jnp.float32),
        ],
    )
    def k(xyzT_hbm, idx_hbm, dist_hbm, out_hbm, xyz_v, idx_v, f4_v):
        wid = lax.axis_index("s") * 2 + lax.axis_index("c")
        bl = wid // WPB
        k0 = (wid % WPB) * KPW
        pltpu.sync_copy(xyzT_hbm.at[b0 + bl], xyz_v)
        for dk in range(KPW):
            kk = k0 + dk
            pltpu.sync_copy(idx_hbm.at[b0 + bl, kk], idx_v)
            pltpu.sync_copy(dist_hbm.at[b0 + bl, kk], f4_v.at[pl.ds(3 * N, N)])

            def body(i, carry):
                iv = idx_v[pl.ds(i * 16, 16)]
                for c in range(3):
                    g = plsc.load_gather(xyz_v, [iv + (c * N)])
                    f4_v[pl.ds(c * N + i * 16, 16)] = g
                return carry

            lax.fori_loop(0, NV, body, 0)
            pltpu.sync_copy(f4_v, out_hbm.at[bl, kk])

    return k(xyzTf, idxP, distP)


def _cct_body(xyz_ref, at_ref, b1_ref, out_ref):
    out_ref[0] = jnp.dot(at_ref[...], xyz_ref[0],
                         preferred_element_type=jnp.float32) + b1_ref[...]


def _cct(xyzT, AT, b1col):
    B, _, N = xyzT.shape
    dim = AT.shape[0]
    return pl.pallas_call(
        _cct_body,
        grid=(B,),
        in_specs=[
            pl.BlockSpec((1, 3, N), lambda b: (b, 0, 0)),
            pl.BlockSpec((dim, 3), lambda b: (0, 0)),
            pl.BlockSpec((dim, 1), lambda b: (0, 0)),
        ],
        out_specs=pl.BlockSpec((1, dim, N), lambda b: (b, 0, 0)),
        out_shape=jax.ShapeDtypeStruct((B, dim, N), jnp.float32),
    )(xyzT, AT, b1col)


_KB = 8


def _mlp_body(f4_ref, cct_ref, wn_ref, w2_ref, b2_ref, out_ref):
    cct = cct_ref[0]
    for j in range(_KB):
        ht = jnp.dot(wn_ref[...], f4_ref[0, j],
                     preferred_element_type=jnp.float32) + cct
        ht = jnp.maximum(ht, 0.0).astype(jnp.bfloat16)
        out_ref[0, j] = jnp.dot(w2_ref[...], ht,
                                preferred_element_type=jnp.float32) + b2_ref[...]


def _mlp_alias_body(f4_ref, cct_ref, wn_ref, w2_ref, b2_ref, buf_ref, out_ref):
    _mlp_body(f4_ref, cct_ref, wn_ref, w2_ref, b2_ref, out_ref)


def _mlp(F4h, CcT, Wxn4T, W2T, b2col, b0, buf=None):
    bh, K, N4 = F4h.shape
    N = N4 // 4
    B = CcT.shape[0]
    dim = W2T.shape[0]
    F4h = F4h.reshape(bh, K, 4, N)
    in_specs = [
        pl.BlockSpec((1, _KB, 4, N), lambda b, k: (b, k, 0, 0)),
        pl.BlockSpec((1, dim, N), lambda b, k: (b0 + b, 0, 0)),
        pl.BlockSpec((dim, 4), lambda b, k: (0, 0)),
        pl.BlockSpec((dim, dim), lambda b, k: (0, 0)),
        pl.BlockSpec((dim, 1), lambda b, k: (0, 0)),
    ]
    args = [F4h, CcT, Wxn4T, W2T, b2col]
    kwargs = {}
    body = _mlp_body
    if buf is not None:
        in_specs.append(pl.BlockSpec(memory_space=pl.MemorySpace.ANY))
        args.append(buf)
        kwargs["input_output_aliases"] = {5: 0}
        body = _mlp_alias_body
    return pl.pallas_call(
        body,
        grid=(bh, K // _KB),
        in_specs=in_specs,
        out_specs=pl.BlockSpec((1, _KB, dim, N),
                               lambda b, k: (b0 + b, k, 0, 0)),
        out_shape=jax.ShapeDtypeStruct((B, K, dim, N), jnp.float32),
        **kwargs,
    )(*args)


def kernel(xyz, idx, dist, W1, b1, W2, b2, num_neighbors=16):
    B, N, K = idx.shape
    xyzT = jnp.transpose(xyz, (0, 2, 1))
    xyzTf = xyzT.reshape(B, 3 * N)
    idxP = jnp.transpose(idx, (0, 2, 1))
    distP = jnp.transpose(dist, (0, 2, 1))
    A = W1[0:3] + W1[6:9]
    Bm = W1[3:6] - W1[6:9]
    AT = A.T
    Wxn4T = jnp.concatenate([Bm.T, W1[9:10].T], axis=1)
    b1col = b1[:, None]
    b2col = b2[:, None]
    W2T = W2.T.astype(jnp.bfloat16)
    nsplit = 4
    bh = B // nsplit
    F4s = [_sc_gather(xyzTf, idxP, distP, s * bh, bh) for s in range(nsplit)]
    CcT = _cct(xyzT, AT, b1col)
    buf = _mlp(F4s[0], CcT, Wxn4T, W2T, b2col, 0)
    for s in range(1, nsplit):
        buf = _mlp(F4s[s], CcT, Wxn4T, W2T, b2col, s * bh, buf)
    return jnp.transpose(buf, (0, 3, 1, 2))


# speedup vs baseline: 3.9416x; 1.0014x over previous
import functools

import jax
import jax.numpy as jnp
from jax import lax
from jax.experimental import pallas as pl
from jax.experimental.pallas import tpu as pltpu
from jax.experimental.pallas import tpu_sc as plsc

_NW = 32


def _sc_gather(xyzTf, idxP, distP, b0, bh):
    B, N3 = xyzTf.shape
    N = N3 // 3
    K = idxP.shape[1]
    KPW = (bh * K) // _NW
    WPB = _NW // bh
    NV = N // 16
    mesh = plsc.VectorSubcoreMesh(core_axis_name="c", subcore_axis_name="s")

    @functools.partial(
        pl.kernel,
        mesh=mesh,
        compiler_params=pltpu.CompilerParams(needs_layout_passes=False),
        out_type=jax.ShapeDtypeStruct((bh, K, 4 * N), jnp.float32),
        scratch_types=[
            pltpu.VMEM((3 * N,), jnp.float32),
            pltpu.VMEM((N,), jnp.int32),
            pltpu.VMEM((4 * N,), jnp.float32),
        ],
    )
    def k(xyzT_hbm, idx_hbm, dist_hbm, out_hbm, xyz_v, idx_v, f4_v):
        wid = lax.axis_index("s") * 2 + lax.axis_index("c")
        bl = wid // WPB
        k0 = (wid % WPB) * KPW
        pltpu.sync_copy(xyzT_hbm.at[b0 + bl], xyz_v)
        for dk in range(KPW):
            kk = k0 + dk
            pltpu.sync_copy(idx_hbm.at[b0 + bl, kk], idx_v)
            pltpu.sync_copy(dist_hbm.at[b0 + bl, kk], f4_v.at[pl.ds(3 * N, N)])

            def body(i, carry):
                iv = idx_v[pl.ds(i * 16, 16)]
                for c in range(3):
                    g = plsc.load_gather(xyz_v, [iv + (c * N)])
                    f4_v[pl.ds(c * N + i * 16, 16)] = g
                return carry

            lax.fori_loop(0, NV, body, 0)
            pltpu.sync_copy(f4_v, out_hbm.at[bl, kk])

    return k(xyzTf, idxP, distP)


def _cct_body(xyz_ref, at_ref, b1_ref, out_ref):
    out_ref[0] = jnp.dot(at_ref[...], xyz_ref[0],
                         preferred_element_type=jnp.float32) + b1_ref[...]


def _cct(xyzT, AT, b1col):
    B, _, N = xyzT.shape
    dim = AT.shape[0]
    return pl.pallas_call(
        _cct_body,
        grid=(B,),
        in_specs=[
            pl.BlockSpec((1, 3, N), lambda b: (b, 0, 0)),
            pl.BlockSpec((dim, 3), lambda b: (0, 0)),
            pl.BlockSpec((dim, 1), lambda b: (0, 0)),
        ],
        out_specs=pl.BlockSpec((1, dim, N), lambda b: (b, 0, 0)),
        out_shape=jax.ShapeDtypeStruct((B, dim, N), jnp.float32),
    )(xyzT, AT, b1col)


_KB = 16


def _mlp_body(f4_ref, cct_ref, wn_ref, w2_ref, b2_ref, out_ref):
    cct = cct_ref[0]
    for j in range(_KB):
        ht = jnp.dot(wn_ref[...], f4_ref[0, j],
                     preferred_element_type=jnp.float32) + cct
        ht = jnp.maximum(ht, 0.0).astype(jnp.bfloat16)
        out_ref[0, j] = jnp.dot(w2_ref[...], ht,
                                preferred_element_type=jnp.float32) + b2_ref[...]


def _mlp_alias_body(f4_ref, cct_ref, wn_ref, w2_ref, b2_ref, buf_ref, out_ref):
    _mlp_body(f4_ref, cct_ref, wn_ref, w2_ref, b2_ref, out_ref)


def _mlp(F4h, CcT, Wxn4T, W2T, b2col, b0, buf=None):
    bh, K, N4 = F4h.shape
    N = N4 // 4
    B = CcT.shape[0]
    dim = W2T.shape[0]
    F4h = F4h.reshape(bh, K, 4, N)
    in_specs = [
        pl.BlockSpec((1, _KB, 4, N), lambda b, k: (b, k, 0, 0)),
        pl.BlockSpec((1, dim, N), lambda b, k: (b0 + b, 0, 0)),
        pl.BlockSpec((dim, 4), lambda b, k: (0, 0)),
        pl.BlockSpec((dim, dim), lambda b, k: (0, 0)),
        pl.BlockSpec((dim, 1), lambda b, k: (0, 0)),
    ]
    args = [F4h, CcT, Wxn4T, W2T, b2col]
    kwargs = {}
    body = _mlp_body
    if buf is not None:
        in_specs.append(pl.BlockSpec(memory_space=pl.MemorySpace.ANY))
        args.append(buf)
        kwargs["input_output_aliases"] = {5: 0}
        body = _mlp_alias_body
    return pl.pallas_call(
        body,
        grid=(bh, K // _KB),
        in_specs=in_specs,
        out_specs=pl.BlockSpec((1, _KB, dim, N),
                               lambda b, k: (b0 + b, k, 0, 0)),
        out_shape=jax.ShapeDtypeStruct((B, K, dim, N), jnp.float32),
        **kwargs,
    )(*args)


def kernel(xyz, idx, dist, W1, b1, W2, b2, num_neighbors=16):
    B, N, K = idx.shape
    xyzT = jnp.transpose(xyz, (0, 2, 1))
    xyzTf = xyzT.reshape(B, 3 * N)
    idxP = jnp.transpose(idx, (0, 2, 1))
    distP = jnp.transpose(dist, (0, 2, 1))
    A = W1[0:3] + W1[6:9]
    Bm = W1[3:6] - W1[6:9]
    AT = A.T
    Wxn4T = jnp.concatenate([Bm.T, W1[9:10].T], axis=1)
    b1col = b1[:, None]
    b2col = b2[:, None]
    W2T = W2.T.astype(jnp.bfloat16)
    nsplit = 4
    bh = B // nsplit
    F4s = [_sc_gather(xyzTf, idxP, distP, s * bh, bh) for s in range(nsplit)]
    CcT = _cct(xyzT, AT, b1col)
    buf = _mlp(F4s[0], CcT, Wxn4T, W2T, b2col, 0)
    for s in range(1, nsplit):
        buf = _mlp(F4s[s], CcT, Wxn4T, W2T, b2col, s * bh, buf)
    return jnp.transpose(buf, (0, 3, 1, 2))
